# Initial kernel scaffold; baseline (speedup 1.0000x reference)
import jax
import jax.numpy as jnp
from jax.experimental import pallas as pl


def kernel(x, edge_index, batch, W1, b1, W2, b2, W_ih, W_hh, b_ih, b_hh, Wl, bl):
    raise NotImplementedError("write your pallas kernel here")


# speedup vs baseline: 12.8376x; 12.8376x over previous
import functools

import jax
import jax.numpy as jnp
from jax import lax
from jax.experimental import pallas as pl
from jax.experimental.pallas import tpu as pltpu
from jax.experimental.pallas import tpu_sc as plsc

N = 10000
IN_C = 128
HID = 128
OUT_C = 64
E = 320000

NCORES = 2
NSUB = 16
NW = NCORES * NSUB
CHUNK = 128
NCH = 79
EPAD = NW * CHUNK * NCH
NPAD = 10240
ROWS_PER_TILE = NPAD // NSUB
DUMMY = 10016

BM = 640
LSTM_BLK = 400


def _deg_body(dst_hbm, out_hbm, didx, buf_v, deg_sh):
    c = lax.axis_index("c")
    s = lax.axis_index("s")
    w = c * NSUB + s

    def _fill(val):
        def _f(i, _):
            def _g(j, _u):
                buf_v[i, pl.ds(j * 16, 16)] = jnp.full((16,), val, jnp.float32)
                return 0

            lax.fori_loop(0, HID // 16, _g, 0)
            return 0

        lax.fori_loop(0, CHUNK, _f, 0)

    _fill(0.0)
    for q in range(ROWS_PER_TILE // CHUNK):
        pltpu.sync_copy(
            buf_v, deg_sh.at[pl.ds(s * ROWS_PER_TILE + q * CHUNK, CHUNK)])
    _fill(1.0)
    plsc.subcore_barrier()

    def _chunk(k, _):
        base = (w * NCH + k) * CHUNK
        pltpu.sync_copy(dst_hbm.at[pl.ds(base, CHUNK)], didx)
        pltpu.sync_copy(buf_v, deg_sh.at[didx], add=True)
        return 0

    lax.fori_loop(0, NCH, _chunk, 0)
    plsc.subcore_barrier()
    for q in range(ROWS_PER_TILE // CHUNK):
        off = s * ROWS_PER_TILE + q * CHUNK
        pltpu.sync_copy(deg_sh.at[pl.ds(off, CHUNK)], buf_v)
        pltpu.sync_copy(buf_v, out_hbm.at[pl.ds(c * NPAD + off, CHUNK)])


def _edge_scatter_body(src_hbm, dst_hbm, y_hbm, out_hbm, sidx, didx, rows, z_sh, sem):
    c = lax.axis_index("c")
    s = lax.axis_index("s")
    w = c * NSUB + s

    def _zinit(i, _):
        def _zin(j, _u):
            rows[i, pl.ds(j * 16, 16)] = jnp.zeros((16,), jnp.float32)
            return 0

        lax.fori_loop(0, HID // 16, _zin, 0)
        return 0

    lax.fori_loop(0, CHUNK, _zinit, 0)
    for q in range(ROWS_PER_TILE // CHUNK):
        pltpu.sync_copy(
            rows, z_sh.at[pl.ds(s * ROWS_PER_TILE + q * CHUNK, CHUNK)]
        )
    plsc.subcore_barrier()

    def _chunk(k, _):
        base = (w * NCH + k) * CHUNK
        pltpu.sync_copy(src_hbm.at[pl.ds(base, CHUNK)], sidx)
        pltpu.sync_copy(dst_hbm.at[pl.ds(base, CHUNK)], didx)
        pltpu.async_copy(y_hbm.at[sidx], rows, sem).wait()
        pltpu.sync_copy(rows, z_sh.at[didx], add=True)
        return 0

    lax.fori_loop(0, NCH, _chunk, 0)
    plsc.subcore_barrier()
    for q in range(ROWS_PER_TILE // CHUNK):
        off = s * ROWS_PER_TILE + q * CHUNK
        pltpu.sync_copy(z_sh.at[pl.ds(off, CHUNK)], rows)
        pltpu.sync_copy(rows, out_hbm.at[pl.ds(c * NPAD + off, CHUNK)])


@functools.lru_cache(maxsize=1)
def _get_sc_kernels():
    mesh = plsc.VectorSubcoreMesh(core_axis_name="c", subcore_axis_name="s")
    deg_k = pl.kernel(
        _deg_body,
        out_type=jax.ShapeDtypeStruct((NCORES * NPAD, HID), jnp.float32),
        mesh=mesh,
        scratch_types=[
            pltpu.VMEM((CHUNK,), jnp.int32),
            pltpu.VMEM((CHUNK, HID), jnp.float32),
            pltpu.VMEM_SHARED((NPAD, HID), jnp.float32),
        ],
    )
    scat_k = pl.kernel(
        _edge_scatter_body,
        out_type=jax.ShapeDtypeStruct((NCORES * NPAD, HID), jnp.float32),
        mesh=mesh,
        scratch_types=[
            pltpu.VMEM((CHUNK,), jnp.int32),
            pltpu.VMEM((CHUNK,), jnp.int32),
            pltpu.VMEM((CHUNK, HID), jnp.float32),
            pltpu.VMEM_SHARED((NPAD, HID), jnp.float32),
            pltpu.SemaphoreType.DMA,
        ],
    )
    return deg_k, scat_k


def _dinv_block(deg_ref):
    d = deg_ref[0, :, 0:1] + deg_ref[1, :, 0:1] + 1.0
    return lax.rsqrt(d)


def _k1_body(x_ref, w_ref, deg_ref, y_ref):
    xw = jnp.dot(x_ref[...], w_ref[...], preferred_element_type=jnp.float32)
    y_ref[...] = _dinv_block(deg_ref) * xw


def _k2_body(z_ref, y_ref, deg_ref, b_ref, w_ref, o_ref):
    dinv = _dinv_block(deg_ref)
    agg = z_ref[0] + z_ref[1] + y_ref[...]
    h = jnp.maximum(dinv * agg + b_ref[...], 0.0)
    o_ref[...] = dinv * jnp.dot(h, w_ref[...], preferred_element_type=jnp.float32)


def _k3_body(z_ref, y_ref, deg_ref, b_ref, wih_ref, bg_ref, o_ref):
    dinv = _dinv_block(deg_ref)
    agg = z_ref[0] + z_ref[1] + y_ref[...]
    h = jnp.maximum(dinv * agg + b_ref[...], 0.0)
    o_ref[...] = (
        jnp.dot(h, wih_ref[...], preferred_element_type=jnp.float32) + bg_ref[...]
    )


def _lstm_body(ig_ref, whh_ref, wl_ref, bl_ref, o_ref, h_ref, c_ref, hs_ref):
    @pl.when(pl.program_id(0) == 0)
    def _():
        h_ref[...] = jnp.zeros_like(h_ref)
        c_ref[...] = jnp.zeros_like(c_ref)

    whh = whh_ref[...]

    def _step(t, carry):
        h, cc = carry
        gates = ig_ref[pl.ds(t, 1), :] + jnp.dot(
            h, whh, preferred_element_type=jnp.float32
        )
        i = jax.nn.sigmoid(gates[:, 0:HID])
        f = jax.nn.sigmoid(gates[:, HID : 2 * HID])
        g = jnp.tanh(gates[:, 2 * HID : 3 * HID])
        o = jax.nn.sigmoid(gates[:, 3 * HID : 4 * HID])
        cc = f * cc + i * g
        h = o * jnp.tanh(cc)
        hs_ref[pl.ds(t, 1), :] = h
        return (h, cc)

    h, cc = lax.fori_loop(0, LSTM_BLK, _step, (h_ref[...], c_ref[...]))
    h_ref[...] = h
    c_ref[...] = cc
    o_ref[...] = (
        jnp.dot(hs_ref[...], wl_ref[...], preferred_element_type=jnp.float32)
        + bl_ref[...]
    )


def _make_tc_calls():
    nb = NPAD // BM
    k1 = pl.pallas_call(
        _k1_body,
        grid=(nb,),
        in_specs=[
            pl.BlockSpec((BM, IN_C), lambda i: (i, 0)),
            pl.BlockSpec((IN_C, HID), lambda i: (0, 0)),
            pl.BlockSpec((NCORES, BM, HID), lambda i: (0, i, 0)),
        ],
        out_specs=pl.BlockSpec((BM, HID), lambda i: (i, 0)),
        out_shape=jax.ShapeDtypeStruct((NPAD, HID), jnp.float32),
    )
    k2 = pl.pallas_call(
        _k2_body,
        grid=(nb,),
        in_specs=[
            pl.BlockSpec((NCORES, BM, HID), lambda i: (0, i, 0)),
            pl.BlockSpec((BM, HID), lambda i: (i, 0)),
            pl.BlockSpec((NCORES, BM, HID), lambda i: (0, i, 0)),
            pl.BlockSpec((1, HID), lambda i: (0, 0)),
            pl.BlockSpec((HID, HID), lambda i: (0, 0)),
        ],
        out_specs=pl.BlockSpec((BM, HID), lambda i: (i, 0)),
        out_shape=jax.ShapeDtypeStruct((NPAD, HID), jnp.float32),
    )
    k3 = pl.pallas_call(
        _k3_body,
        grid=(nb,),
        in_specs=[
            pl.BlockSpec((NCORES, BM, HID), lambda i: (0, i, 0)),
            pl.BlockSpec((BM, HID), lambda i: (i, 0)),
            pl.BlockSpec((NCORES, BM, HID), lambda i: (0, i, 0)),
            pl.BlockSpec((1, HID), lambda i: (0, 0)),
            pl.BlockSpec((HID, 4 * HID), lambda i: (0, 0)),
            pl.BlockSpec((1, 4 * HID), lambda i: (0, 0)),
        ],
        out_specs=pl.BlockSpec((BM, 4 * HID), lambda i: (i, 0)),
        out_shape=jax.ShapeDtypeStruct((NPAD, 4 * HID), jnp.float32),
    )
    lstm = pl.pallas_call(
        _lstm_body,
        grid=(N // LSTM_BLK,),
        in_specs=[
            pl.BlockSpec((LSTM_BLK, 4 * HID), lambda i: (i, 0)),
            pl.BlockSpec((HID, 4 * HID), lambda i: (0, 0)),
            pl.BlockSpec((HID, OUT_C), lambda i: (0, 0)),
            pl.BlockSpec((1, OUT_C), lambda i: (0, 0)),
        ],
        out_specs=pl.BlockSpec((LSTM_BLK, OUT_C), lambda i: (i, 0)),
        out_shape=jax.ShapeDtypeStruct((N, OUT_C), jnp.float32),
        scratch_shapes=[
            pltpu.VMEM((1, HID), jnp.float32),
            pltpu.VMEM((1, HID), jnp.float32),
            pltpu.VMEM((LSTM_BLK, HID), jnp.float32),
        ],
    )
    return k1, k2, k3, lstm


_K1, _K2, _K3, _LSTM = _make_tc_calls()


def kernel(x, edge_index, batch, W1, b1, W2, b2, W_ih, W_hh, b_ih, b_hh, Wl, bl):
    del batch
    pad_e = EPAD - E
    src = jnp.concatenate(
        [edge_index[0], jnp.full((pad_e,), DUMMY, jnp.int32)])
    dst = jnp.concatenate(
        [edge_index[1], jnp.full((pad_e,), DUMMY, jnp.int32)])
    x_p = jnp.pad(x, ((0, NPAD - N), (0, 0)))

    deg_k, scat_k = _get_sc_kernels()
    degp = deg_k(dst).reshape(NCORES, NPAD, HID)

    y1 = _K1(x_p, W1, degp)
    z1 = scat_k(src, dst, y1).reshape(NCORES, NPAD, HID)
    y2 = _K2(z1, y1, degp, b1.reshape(1, HID), W2)
    z2 = scat_k(src, dst, y2).reshape(NCORES, NPAD, HID)
    ig = _K3(
        z2, y2, degp, b2.reshape(1, HID),
        W_ih.T, (b_ih + b_hh).reshape(1, 4 * HID),
    )
    out = _LSTM(ig, W_hh.T, Wl, bl.reshape(1, OUT_C))
    return out


# speedup vs baseline: 12.8863x; 1.0038x over previous
import functools

import jax
import jax.numpy as jnp
from jax import lax
from jax.experimental import pallas as pl
from jax.experimental.pallas import tpu as pltpu
from jax.experimental.pallas import tpu_sc as plsc

N = 10000
IN_C = 128
HID = 128
OUT_C = 64
E = 320000

NCORES = 2
NSUB = 16
NW = NCORES * NSUB
CHUNK = 128
NCH = 79
EPAD = NW * CHUNK * NCH
NPAD = 10240
ROWS_PER_TILE = NPAD // NSUB
DUMMY = 10016

BM = 640
LSTM_BLK = 400


def _deg_body(dst_hbm, out_hbm, didx, buf_v, deg_sh):
    c = lax.axis_index("c")
    s = lax.axis_index("s")
    w = c * NSUB + s

    def _fill(val):
        def _f(i, _):
            def _g(j, _u):
                buf_v[i, pl.ds(j * 16, 16)] = jnp.full((16,), val, jnp.float32)
                return 0

            lax.fori_loop(0, HID // 16, _g, 0)
            return 0

        lax.fori_loop(0, CHUNK, _f, 0)

    _fill(0.0)
    for q in range(ROWS_PER_TILE // CHUNK):
        pltpu.sync_copy(
            buf_v, deg_sh.at[pl.ds(s * ROWS_PER_TILE + q * CHUNK, CHUNK)])
    _fill(1.0)
    plsc.subcore_barrier()

    def _chunk(k, _):
        base = (w * NCH + k) * CHUNK
        pltpu.sync_copy(dst_hbm.at[pl.ds(base, CHUNK)], didx)
        pltpu.sync_copy(buf_v, deg_sh.at[didx], add=True)
        return 0

    lax.fori_loop(0, NCH, _chunk, 0)
    plsc.subcore_barrier()
    for q in range(ROWS_PER_TILE // CHUNK):
        off = s * ROWS_PER_TILE + q * CHUNK
        pltpu.sync_copy(deg_sh.at[pl.ds(off, CHUNK)], buf_v)
        pltpu.sync_copy(buf_v, out_hbm.at[pl.ds(c * NPAD + off, CHUNK)])


def _edge_scatter_body(src_hbm, dst_hbm, y_hbm, out_hbm, sidx, didx, rows, z_sh, sem):
    c = lax.axis_index("c")
    s = lax.axis_index("s")
    w = c * NSUB + s

    def _zinit(i, _):
        def _zin(j, _u):
            rows[i, pl.ds(j * 16, 16)] = jnp.zeros((16,), jnp.float32)
            return 0

        lax.fori_loop(0, HID // 16, _zin, 0)
        return 0

    lax.fori_loop(0, CHUNK, _zinit, 0)
    for q in range(ROWS_PER_TILE // CHUNK):
        pltpu.sync_copy(
            rows, z_sh.at[pl.ds(s * ROWS_PER_TILE + q * CHUNK, CHUNK)]
        )
    plsc.subcore_barrier()

    def _chunk(k, _):
        base = (w * NCH + k) * CHUNK
        pltpu.sync_copy(src_hbm.at[pl.ds(base, CHUNK)], sidx)
        pltpu.sync_copy(dst_hbm.at[pl.ds(base, CHUNK)], didx)
        pltpu.async_copy(y_hbm.at[sidx], rows, sem).wait()
        pltpu.sync_copy(rows, z_sh.at[didx], add=True)
        return 0

    lax.fori_loop(0, NCH, _chunk, 0)
    plsc.subcore_barrier()
    for q in range(ROWS_PER_TILE // CHUNK):
        off = s * ROWS_PER_TILE + q * CHUNK
        pltpu.sync_copy(z_sh.at[pl.ds(off, CHUNK)], rows)
        pltpu.sync_copy(rows, out_hbm.at[pl.ds(c * NPAD + off, CHUNK)])


@functools.lru_cache(maxsize=1)
def _get_sc_kernels():
    mesh = plsc.VectorSubcoreMesh(core_axis_name="c", subcore_axis_name="s")
    deg_k = pl.kernel(
        _deg_body,
        out_type=jax.ShapeDtypeStruct((NCORES * NPAD, HID), jnp.float32),
        mesh=mesh,
        scratch_types=[
            pltpu.VMEM((CHUNK,), jnp.int32),
            pltpu.VMEM((CHUNK, HID), jnp.float32),
            pltpu.VMEM_SHARED((NPAD, HID), jnp.float32),
        ],
    )
    scat_k = pl.kernel(
        _edge_scatter_body,
        out_type=jax.ShapeDtypeStruct((NCORES * NPAD, HID), jnp.float32),
        mesh=mesh,
        scratch_types=[
            pltpu.VMEM((CHUNK,), jnp.int32),
            pltpu.VMEM((CHUNK,), jnp.int32),
            pltpu.VMEM((CHUNK, HID), jnp.float32),
            pltpu.VMEM_SHARED((NPAD, HID), jnp.float32),
            pltpu.SemaphoreType.DMA,
        ],
    )
    return deg_k, scat_k


def _dinv_block(deg_ref):
    d = deg_ref[0, :, 0:1] + deg_ref[1, :, 0:1] + 1.0
    return lax.rsqrt(d)


def _k1_body(x_ref, w_ref, deg_ref, y_ref):
    xw = jnp.dot(x_ref[...], w_ref[...], preferred_element_type=jnp.float32)
    y_ref[...] = _dinv_block(deg_ref) * xw


def _k2_body(z_ref, y_ref, deg_ref, b_ref, w_ref, o_ref):
    dinv = _dinv_block(deg_ref)
    agg = z_ref[0] + z_ref[1] + y_ref[...]
    h = jnp.maximum(dinv * agg + b_ref[...], 0.0)
    o_ref[...] = dinv * jnp.dot(h, w_ref[...], preferred_element_type=jnp.float32)


def _k3_body(z_ref, y_ref, deg_ref, b_ref, wih_ref, bg_ref, o_ref):
    dinv = _dinv_block(deg_ref)
    agg = z_ref[0] + z_ref[1] + y_ref[...]
    h = jnp.maximum(dinv * agg + b_ref[...], 0.0)
    o_ref[...] = (
        jnp.dot(h, wih_ref[...], preferred_element_type=jnp.float32) + bg_ref[...]
    )


def _lstm_body(ig_ref, whh_ref, wl_ref, bl_ref, o_ref, h_ref, c_ref, hs_ref):
    @pl.when(pl.program_id(0) == 0)
    def _():
        h_ref[...] = jnp.zeros_like(h_ref)
        c_ref[...] = jnp.zeros_like(c_ref)

    whh = whh_ref[...]

    def _step(t, carry):
        h, cc = carry
        gates = ig_ref[pl.ds(t, 1), :] + jnp.dot(
            h.astype(jnp.bfloat16), whh, preferred_element_type=jnp.float32
        )
        i = jax.nn.sigmoid(gates[:, 0:HID])
        f = jax.nn.sigmoid(gates[:, HID : 2 * HID])
        g = jnp.tanh(gates[:, 2 * HID : 3 * HID])
        o = jax.nn.sigmoid(gates[:, 3 * HID : 4 * HID])
        cc = f * cc + i * g
        h = o * jnp.tanh(cc)
        hs_ref[pl.ds(t, 1), :] = h
        return (h, cc)

    h, cc = lax.fori_loop(0, LSTM_BLK, _step, (h_ref[...], c_ref[...]))
    h_ref[...] = h
    c_ref[...] = cc
    o_ref[...] = (
        jnp.dot(hs_ref[...], wl_ref[...], preferred_element_type=jnp.float32)
        + bl_ref[...]
    )


def _make_tc_calls():
    nb = NPAD // BM
    k1 = pl.pallas_call(
        _k1_body,
        grid=(nb,),
        in_specs=[
            pl.BlockSpec((BM, IN_C), lambda i: (i, 0)),
            pl.BlockSpec((IN_C, HID), lambda i: (0, 0)),
            pl.BlockSpec((NCORES, BM, HID), lambda i: (0, i, 0)),
        ],
        out_specs=pl.BlockSpec((BM, HID), lambda i: (i, 0)),
        out_shape=jax.ShapeDtypeStruct((NPAD, HID), jnp.float32),
    )
    k2 = pl.pallas_call(
        _k2_body,
        grid=(nb,),
        in_specs=[
            pl.BlockSpec((NCORES, BM, HID), lambda i: (0, i, 0)),
            pl.BlockSpec((BM, HID), lambda i: (i, 0)),
            pl.BlockSpec((NCORES, BM, HID), lambda i: (0, i, 0)),
            pl.BlockSpec((1, HID), lambda i: (0, 0)),
            pl.BlockSpec((HID, HID), lambda i: (0, 0)),
        ],
        out_specs=pl.BlockSpec((BM, HID), lambda i: (i, 0)),
        out_shape=jax.ShapeDtypeStruct((NPAD, HID), jnp.float32),
    )
    k3 = pl.pallas_call(
        _k3_body,
        grid=(nb,),
        in_specs=[
            pl.BlockSpec((NCORES, BM, HID), lambda i: (0, i, 0)),
            pl.BlockSpec((BM, HID), lambda i: (i, 0)),
            pl.BlockSpec((NCORES, BM, HID), lambda i: (0, i, 0)),
            pl.BlockSpec((1, HID), lambda i: (0, 0)),
            pl.BlockSpec((HID, 4 * HID), lambda i: (0, 0)),
            pl.BlockSpec((1, 4 * HID), lambda i: (0, 0)),
        ],
        out_specs=pl.BlockSpec((BM, 4 * HID), lambda i: (i, 0)),
        out_shape=jax.ShapeDtypeStruct((NPAD, 4 * HID), jnp.float32),
    )
    lstm = pl.pallas_call(
        _lstm_body,
        grid=(N // LSTM_BLK,),
        in_specs=[
            pl.BlockSpec((LSTM_BLK, 4 * HID), lambda i: (i, 0)),
            pl.BlockSpec((HID, 4 * HID), lambda i: (0, 0)),
            pl.BlockSpec((HID, OUT_C), lambda i: (0, 0)),
            pl.BlockSpec((1, OUT_C), lambda i: (0, 0)),
        ],
        out_specs=pl.BlockSpec((LSTM_BLK, OUT_C), lambda i: (i, 0)),
        out_shape=jax.ShapeDtypeStruct((N, OUT_C), jnp.float32),
        scratch_shapes=[
            pltpu.VMEM((1, HID), jnp.float32),
            pltpu.VMEM((1, HID), jnp.float32),
            pltpu.VMEM((LSTM_BLK, HID), jnp.float32),
        ],
    )
    return k1, k2, k3, lstm


_K1, _K2, _K3, _LSTM = _make_tc_calls()


def kernel(x, edge_index, batch, W1, b1, W2, b2, W_ih, W_hh, b_ih, b_hh, Wl, bl):
    del batch
    pad_e = EPAD - E
    src = jnp.concatenate(
        [edge_index[0], jnp.full((pad_e,), DUMMY, jnp.int32)])
    dst = jnp.concatenate(
        [edge_index[1], jnp.full((pad_e,), DUMMY, jnp.int32)])
    x_p = jnp.pad(x, ((0, NPAD - N), (0, 0)))

    deg_k, scat_k = _get_sc_kernels()
    degp = deg_k(dst).reshape(NCORES, NPAD, HID)

    y1 = _K1(x_p, W1, degp)
    z1 = scat_k(src, dst, y1).reshape(NCORES, NPAD, HID)
    y2 = _K2(z1, y1, degp, b1.reshape(1, HID), W2)
    z2 = scat_k(src, dst, y2).reshape(NCORES, NPAD, HID)
    ig = _K3(
        z2, y2, degp, b2.reshape(1, HID),
        W_ih.T, (b_ih + b_hh).reshape(1, 4 * HID),
    )
    out = _LSTM(ig, W_hh.T.astype(jnp.bfloat16), Wl, bl.reshape(1, OUT_C))
    return out


# speedup vs baseline: 15.0768x; 1.1700x over previous
import functools

import jax
import jax.numpy as jnp
from jax import lax
from jax.experimental import pallas as pl
from jax.experimental.pallas import tpu as pltpu
from jax.experimental.pallas import tpu_sc as plsc

N = 10000
IN_C = 128
HID = 128
OUT_C = 64
E = 320000

NCORES = 2
NSUB = 16
NW = NCORES * NSUB
CHUNK = 64
NCH = 158
EPAD = NW * CHUNK * NCH
NPAD = 10240
ROWS_PER_TILE = NPAD // NSUB
DUMMY = 10016

BM = 640
LSTM_BLK = 400


def _deg_body(dst_hbm, out_hbm, didx, buf_v, deg_sh):
    c = lax.axis_index("c")
    s = lax.axis_index("s")
    w = c * NSUB + s

    def _fill(val):
        def _f(i, _):
            def _g(j, _u):
                buf_v[i, pl.ds(j * 16, 16)] = jnp.full((16,), val, jnp.float32)
                return 0

            lax.fori_loop(0, HID // 16, _g, 0)
            return 0

        lax.fori_loop(0, CHUNK, _f, 0)

    _fill(0.0)
    for q in range(ROWS_PER_TILE // CHUNK):
        pltpu.sync_copy(
            buf_v, deg_sh.at[pl.ds(s * ROWS_PER_TILE + q * CHUNK, CHUNK)])
    _fill(1.0)
    plsc.subcore_barrier()

    def _chunk(k, _):
        pltpu.sync_copy(dst_hbm.at[pl.ds((w * NCH + k) * CHUNK, CHUNK)], didx)
        pltpu.sync_copy(buf_v, deg_sh.at[didx], add=True)
        return 0

    lax.fori_loop(0, NCH, _chunk, 0)
    plsc.subcore_barrier()
    for q in range(ROWS_PER_TILE // CHUNK):
        off = s * ROWS_PER_TILE + q * CHUNK
        pltpu.sync_copy(deg_sh.at[pl.ds(off, CHUNK)], buf_v)
        pltpu.sync_copy(buf_v, out_hbm.at[pl.ds(c * NPAD + off, CHUNK)])


def _edge_scatter_body(src_hbm, dst_hbm, y_hbm, out_hbm, sidx_all,
                       didx_a, didx_b, rows_a, rows_b, z_sh,
                       semg_a, semg_b, sems_a, sems_b, semi_a, semi_b):
    c = lax.axis_index("c")
    s = lax.axis_index("s")
    w = c * NSUB + s

    pltpu.sync_copy(src_hbm.at[pl.ds(w * NCH * CHUNK, NCH * CHUNK)], sidx_all)

    def _zinit(i, _):
        def _zin(j, _u):
            rows_a[i, pl.ds(j * 16, 16)] = jnp.zeros((16,), jnp.float32)
            return 0

        lax.fori_loop(0, HID // 16, _zin, 0)
        return 0

    lax.fori_loop(0, CHUNK, _zinit, 0)

    if True:
        for q in range(ROWS_PER_TILE // CHUNK):
            pltpu.sync_copy(
                rows_a, z_sh.at[pl.ds(s * ROWS_PER_TILE + q * CHUNK, CHUNK)]
            )
        plsc.subcore_barrier()

        def _sidx(k):
            return sidx_all.at[pl.ds(k * CHUNK, CHUNK)]

        def _dslice(k):
            return dst_hbm.at[pl.ds((w * NCH + k) * CHUNK, CHUNK)]

        def _gst(k, rows, sem):
            pltpu.async_copy(y_hbm.at[_sidx(k)], rows, sem)

        def _gwt(k, rows, sem):
            pltpu.make_async_copy(y_hbm.at[_sidx(k)], rows, sem).wait()

        def _sst(k, rows, didx, sem):
            pltpu.async_copy(rows, z_sh.at[didx], sem, add=True)

        def _swt(k, rows, didx, sem):
            pltpu.make_async_copy(rows, z_sh.at[didx], sem).wait()

        def _ist(k, didx, sem):
            pltpu.async_copy(_dslice(k), didx, sem)

        def _iwt(k, didx, sem):
            pltpu.make_async_copy(_dslice(k), didx, sem).wait()

        _ist(0, didx_a, semi_a)
        _gst(0, rows_a, semg_a)

        def _pair(p, _):
            k0 = 2 * p
            _gwt(k0, rows_a, semg_a)

            @pl.when(p > 0)
            def _():
                _swt(k0 - 1, rows_b, didx_b, sems_b)

            _ist(k0 + 1, didx_b, semi_b)
            _gst(k0 + 1, rows_b, semg_b)
            _iwt(k0, didx_a, semi_a)
            _sst(k0, rows_a, didx_a, sems_a)

            _gwt(k0 + 1, rows_b, semg_b)
            _swt(k0, rows_a, didx_a, sems_a)

            @pl.when(k0 + 2 < NCH)
            def _():
                _ist(k0 + 2, didx_a, semi_a)
                _gst(k0 + 2, rows_a, semg_a)

            _iwt(k0 + 1, didx_b, semi_b)
            _sst(k0 + 1, rows_b, didx_b, sems_b)
            return 0

        lax.fori_loop(0, NCH // 2, _pair, 0)
        _swt(NCH - 1, rows_b, didx_b, sems_b)
        plsc.subcore_barrier()
        for q in range(ROWS_PER_TILE // CHUNK):
            off = s * ROWS_PER_TILE + q * CHUNK
            pltpu.sync_copy(z_sh.at[pl.ds(off, CHUNK)], rows_a)
            pltpu.sync_copy(rows_a, out_hbm.at[pl.ds(c * NPAD + off, CHUNK)])


@functools.lru_cache(maxsize=1)
def _get_sc_kernels():
    mesh = plsc.VectorSubcoreMesh(core_axis_name="c", subcore_axis_name="s")
    deg_k = pl.kernel(
        _deg_body,
        out_type=jax.ShapeDtypeStruct((NCORES * NPAD, HID), jnp.float32),
        mesh=mesh,
        scratch_types=[
            pltpu.VMEM((CHUNK,), jnp.int32),
            pltpu.VMEM((CHUNK, HID), jnp.float32),
            pltpu.VMEM_SHARED((NPAD, HID), jnp.float32),
        ],
    )
    scat_k = pl.kernel(
        _edge_scatter_body,
        out_type=jax.ShapeDtypeStruct((NCORES * NPAD, HID), jnp.float32),
        mesh=mesh,
        scratch_types=[
            pltpu.VMEM((NCH * CHUNK,), jnp.int32),
            pltpu.VMEM((CHUNK,), jnp.int32),
            pltpu.VMEM((CHUNK,), jnp.int32),
            pltpu.VMEM((CHUNK, HID), jnp.float32),
            pltpu.VMEM((CHUNK, HID), jnp.float32),
            pltpu.VMEM_SHARED((NPAD, HID), jnp.float32),
            pltpu.SemaphoreType.DMA,
            pltpu.SemaphoreType.DMA,
            pltpu.SemaphoreType.DMA,
            pltpu.SemaphoreType.DMA,
            pltpu.SemaphoreType.DMA,
            pltpu.SemaphoreType.DMA,
        ],
    )
    return deg_k, scat_k


def _dinv_block(deg_ref):
    d = deg_ref[0, :, 0:1] + deg_ref[1, :, 0:1] + 1.0
    return lax.rsqrt(d)


def _k1_body(x_ref, w_ref, deg_ref, y_ref):
    xw = jnp.dot(x_ref[...], w_ref[...], preferred_element_type=jnp.float32)
    y_ref[...] = _dinv_block(deg_ref) * xw


def _k2_body(z_ref, y_ref, deg_ref, b_ref, w_ref, o_ref):
    dinv = _dinv_block(deg_ref)
    agg = z_ref[0] + z_ref[1] + y_ref[...]
    h = jnp.maximum(dinv * agg + b_ref[...], 0.0)
    o_ref[...] = dinv * jnp.dot(h, w_ref[...], preferred_element_type=jnp.float32)


def _k3_body(z_ref, y_ref, deg_ref, b_ref, wih_ref, bg_ref, o_ref):
    dinv = _dinv_block(deg_ref)
    agg = z_ref[0] + z_ref[1] + y_ref[...]
    h = jnp.maximum(dinv * agg + b_ref[...], 0.0)
    o_ref[...] = (
        jnp.dot(h, wih_ref[...], preferred_element_type=jnp.float32) + bg_ref[...]
    )


def _lstm_body(ig_ref, whh_ref, wl_ref, bl_ref, o_ref, h_ref, c_ref, hs_ref):
    @pl.when(pl.program_id(0) == 0)
    def _():
        h_ref[...] = jnp.zeros_like(h_ref)
        c_ref[...] = jnp.zeros_like(c_ref)

    whh = whh_ref[...]

    def _step(t, carry):
        h, cc = carry
        gates = ig_ref[pl.ds(t, 1), :] + jnp.dot(
            h.astype(jnp.bfloat16), whh, preferred_element_type=jnp.float32
        )
        i = jax.nn.sigmoid(gates[:, 0:HID])
        f = jax.nn.sigmoid(gates[:, HID : 2 * HID])
        g = jnp.tanh(gates[:, 2 * HID : 3 * HID])
        o = jax.nn.sigmoid(gates[:, 3 * HID : 4 * HID])
        cc = f * cc + i * g
        h = o * jnp.tanh(cc)
        hs_ref[pl.ds(t, 1), :] = h
        return (h, cc)

    h, cc = lax.fori_loop(0, LSTM_BLK, _step, (h_ref[...], c_ref[...]))
    h_ref[...] = h
    c_ref[...] = cc
    o_ref[...] = (
        jnp.dot(hs_ref[...], wl_ref[...], preferred_element_type=jnp.float32)
        + bl_ref[...]
    )


def _make_tc_calls():
    nb = NPAD // BM
    k1 = pl.pallas_call(
        _k1_body,
        grid=(nb,),
        in_specs=[
            pl.BlockSpec((BM, IN_C), lambda i: (i, 0)),
            pl.BlockSpec((IN_C, HID), lambda i: (0, 0)),
            pl.BlockSpec((NCORES, BM, HID), lambda i: (0, i, 0)),
        ],
        out_specs=pl.BlockSpec((BM, HID), lambda i: (i, 0)),
        out_shape=jax.ShapeDtypeStruct((NPAD, HID), jnp.float32),
    )
    k2 = pl.pallas_call(
        _k2_body,
        grid=(nb,),
        in_specs=[
            pl.BlockSpec((NCORES, BM, HID), lambda i: (0, i, 0)),
            pl.BlockSpec((BM, HID), lambda i: (i, 0)),
            pl.BlockSpec((NCORES, BM, HID), lambda i: (0, i, 0)),
            pl.BlockSpec((1, HID), lambda i: (0, 0)),
            pl.BlockSpec((HID, HID), lambda i: (0, 0)),
        ],
        out_specs=pl.BlockSpec((BM, HID), lambda i: (i, 0)),
        out_shape=jax.ShapeDtypeStruct((NPAD, HID), jnp.float32),
    )
    k3 = pl.pallas_call(
        _k3_body,
        grid=(nb,),
        in_specs=[
            pl.BlockSpec((NCORES, BM, HID), lambda i: (0, i, 0)),
            pl.BlockSpec((BM, HID), lambda i: (i, 0)),
            pl.BlockSpec((NCORES, BM, HID), lambda i: (0, i, 0)),
            pl.BlockSpec((1, HID), lambda i: (0, 0)),
            pl.BlockSpec((HID, 4 * HID), lambda i: (0, 0)),
            pl.BlockSpec((1, 4 * HID), lambda i: (0, 0)),
        ],
        out_specs=pl.BlockSpec((BM, 4 * HID), lambda i: (i, 0)),
        out_shape=jax.ShapeDtypeStruct((NPAD, 4 * HID), jnp.float32),
    )
    lstm = pl.pallas_call(
        _lstm_body,
        grid=(N // LSTM_BLK,),
        in_specs=[
            pl.BlockSpec((LSTM_BLK, 4 * HID), lambda i: (i, 0)),
            pl.BlockSpec((HID, 4 * HID), lambda i: (0, 0)),
            pl.BlockSpec((HID, OUT_C), lambda i: (0, 0)),
            pl.BlockSpec((1, OUT_C), lambda i: (0, 0)),
        ],
        out_specs=pl.BlockSpec((LSTM_BLK, OUT_C), lambda i: (i, 0)),
        out_shape=jax.ShapeDtypeStruct((N, OUT_C), jnp.float32),
        scratch_shapes=[
            pltpu.VMEM((1, HID), jnp.float32),
            pltpu.VMEM((1, HID), jnp.float32),
            pltpu.VMEM((LSTM_BLK, HID), jnp.float32),
        ],
    )
    return k1, k2, k3, lstm


_K1, _K2, _K3, _LSTM = _make_tc_calls()


def kernel(x, edge_index, batch, W1, b1, W2, b2, W_ih, W_hh, b_ih, b_hh, Wl, bl):
    del batch
    pad_e = EPAD - E
    fill = DUMMY + (jnp.arange(pad_e, dtype=jnp.int32) % (NPAD - DUMMY))
    src = jnp.concatenate([edge_index[0], fill])
    dst = jnp.concatenate([edge_index[1], fill])
    x_p = jnp.pad(x, ((0, NPAD - N), (0, 0)))

    deg_k, scat_k = _get_sc_kernels()
    degp = deg_k(dst).reshape(NCORES, NPAD, HID)

    y1 = _K1(x_p, W1, degp)
    z1 = scat_k(src, dst, y1).reshape(NCORES, NPAD, HID)
    y2 = _K2(z1, y1, degp, b1.reshape(1, HID), W2)
    z2 = scat_k(src, dst, y2).reshape(NCORES, NPAD, HID)
    ig = _K3(
        z2, y2, degp, b2.reshape(1, HID),
        W_ih.T, (b_ih + b_hh).reshape(1, 4 * HID),
    )
    out = _LSTM(ig, W_hh.T.astype(jnp.bfloat16), Wl, bl.reshape(1, OUT_C))
    return out
